# baseline (speedup 1.0000x reference)
import functools

import jax
import jax.numpy as jnp
from jax import lax
from jax.experimental import pallas as pl
from jax.experimental.pallas import tpu as pltpu
from jax.experimental.pallas import tpu_sc as plsc

N = 10000
NP = 10240
E_RAW = 160000
E_TOT = E_RAW + N
EP = 170496
NC, NS, L = 2, 16, 16
NW = NC * NS
EPW = EP // NW
NPW = NP // NW
RB = 256
NRB = NP // RB
NP4 = NP * 4

_mesh_cache = []


def _mesh():
    if not _mesh_cache:
        _mesh_cache.append(plsc.VectorSubcoreMesh(
            core_axis_name="c", subcore_axis_name="s",
            num_cores=NC, num_subcores=NS))
    return _mesh_cache[0]


def _f32(*shape):
    return jax.ShapeDtypeStruct(shape, jnp.float32)


_SC_PARAMS = dict(compiler_params=pltpu.CompilerParams(
    needs_layout_passes=False))


def _mm_xp_body(act, ht_ref, wt_ref, as_ref, ad_ref, b_ref, xpt_ref,
                als_ref, ald_ref):
    ht = ht_ref[...]
    if act == "relu":
        ht = jnp.maximum(ht + b_ref[...], 0.0)
    xpt = jnp.dot(wt_ref[...], ht, preferred_element_type=jnp.float32)
    xpt_ref[...] = xpt
    als_ref[0, 0, :] = jnp.dot(as_ref[...], xpt,
                               preferred_element_type=jnp.float32)[0]
    ald_ref[0, 0, :] = jnp.dot(ad_ref[...], xpt,
                               preferred_element_type=jnp.float32)[0]


def _mm_xp(ht, WT, a_s, a_d, bias, act):
    d_out, d_in = WT.shape
    out = pl.pallas_call(
        functools.partial(_mm_xp_body, act),
        grid=(NRB,),
        in_specs=[
            pl.BlockSpec((d_in, RB), lambda i: (0, i)),
            pl.BlockSpec((d_out, d_in), lambda i: (0, 0)),
            pl.BlockSpec((1, d_out), lambda i: (0, 0)),
            pl.BlockSpec((1, d_out), lambda i: (0, 0)),
            pl.BlockSpec((d_in, 1), lambda i: (0, 0)),
        ],
        out_specs=[
            pl.BlockSpec((d_out, RB), lambda i: (0, i)),
            pl.BlockSpec((1, 1, RB), lambda i: (i, 0, 0)),
            pl.BlockSpec((1, 1, RB), lambda i: (i, 0, 0)),
        ],
        out_shape=[_f32(d_out, NP), _f32(NRB, 1, RB), _f32(NRB, 1, RB)],
    )(ht, WT, a_s[None, :], a_d[None, :], bias[:, None])
    xpt, als3, ald3 = out
    return xpt, als3.reshape(NP), ald3.reshape(NP)


def _mm_lin_body(hp_ref, ob_ref, bb_ref, w1_ref, w2_ref, lb_ref, o_ref):
    t = jnp.tanh(ob_ref[...] + bb_ref[...])
    acc = jnp.dot(w1_ref[...], hp_ref[...], preferred_element_type=jnp.float32)
    acc += jnp.dot(w2_ref[...], t, preferred_element_type=jnp.float32)
    o_ref[...] = jnp.maximum(acc + lb_ref[...], 0.0)


def _mm_lin(hp_t, ob_t, bias_b, W1, W2, lin_b):
    hl, dp = W1.shape
    d = W2.shape[1]
    return pl.pallas_call(
        _mm_lin_body,
        grid=(NRB,),
        in_specs=[
            pl.BlockSpec((dp, RB), lambda i: (0, i)),
            pl.BlockSpec((d, RB), lambda i: (0, i)),
            pl.BlockSpec((d, 1), lambda i: (0, 0)),
            pl.BlockSpec((hl, dp), lambda i: (0, 0)),
            pl.BlockSpec((hl, d), lambda i: (0, 0)),
            pl.BlockSpec((hl, 1), lambda i: (0, 0)),
        ],
        out_specs=pl.BlockSpec((hl, RB), lambda i: (0, i)),
        out_shape=_f32(hl, NP),
    )(hp_t, ob_t, bias_b[:, None], W1, W2, lin_b[:, None])


def _mm_final_body(ht_ref, wt_ref, b_ref, o_ref):
    col = ht_ref[:, pl.ds(14, 1)]
    o_ref[0, :] = jnp.sum(wt_ref[...] * col, axis=0) + b_ref[0, :]


def _mm_final(ht, WoutT, b_out):
    hl, o = WoutT.shape
    out = pl.pallas_call(
        _mm_final_body,
        grid=(1,),
        in_specs=[
            pl.BlockSpec((hl, 128), lambda i: (0, 0)),
            pl.BlockSpec((hl, o), lambda i: (0, 0)),
            pl.BlockSpec((1, o), lambda i: (0, 0)),
        ],
        out_specs=pl.BlockSpec((1, o), lambda i: (0, 0)),
        out_shape=_f32(1, o),
    )(ht, WoutT, b_out[None, :])
    return out[0]


def _k1_body(als_h, ald_h, pk_h, ex_h, dp_h,
             als_v, ald_v, pk_v, ex_v, dp_v):
    c = lax.axis_index("c")
    s = lax.axis_index("s")
    wid = s * NC + c
    base = wid * EPW
    pltpu.sync_copy(als_h, als_v)
    pltpu.sync_copy(ald_h, ald_v)
    pltpu.sync_copy(pk_h.at[pl.ds(base, EPW)], pk_v)

    zero = jnp.zeros((L,), jnp.float32)

    def zbody(i, _):
        dp_v[pl.ds(i * L, L)] = zero
        return 0
    lax.fori_loop(0, NP // L, zbody, 0)

    def ebody(g, _):
        sl = pl.ds(g * L, L)
        pk = pk_v[sl]
        s16 = lax.shift_right_logical(jnp.bitwise_and(pk, 65535), 2)
        d16 = lax.shift_right_logical(pk, 18)
        e = (plsc.load_gather(als_v, [s16])
             + plsc.load_gather(ald_v, [d16]))
        e = jnp.where(e >= 0.0, e, e * 0.2)
        ex = jnp.exp(e)
        ex_v[sl] = ex
        plsc.addupdate_scatter(dp_v, [d16], ex)
        return 0
    lax.fori_loop(0, EPW // L, ebody, 0)

    pltpu.sync_copy(ex_v, ex_h.at[pl.ds(base, EPW)])
    pltpu.sync_copy(dp_v, dp_h.at[wid])


def _k1(als, ald, pk):
    return pl.kernel(
        _k1_body,
        out_type=(_f32(EP), _f32(NW, NP)),
        mesh=_mesh(),
        scratch_types=[
            pltpu.VMEM((NP,), jnp.float32),
            pltpu.VMEM((NP,), jnp.float32),
            pltpu.VMEM((EPW,), jnp.int32),
            pltpu.VMEM((EPW,), jnp.float32),
            pltpu.VMEM((NP,), jnp.float32),
        ],
        **_SC_PARAMS,
    )(als, ald, pk)


def _k1b_body(dpf_h, rden_h, buf_v, r_v, sem):
    c = lax.axis_index("c")
    s = lax.axis_index("s")
    wid = s * NC + c
    base = wid * NPW
    for half in range(2):
        copies = [
            pltpu.async_copy(
                dpf_h.at[pl.ds((half * 16 + t) * NP + base, NPW)],
                buf_v.at[pl.ds((half * 16 + t) * NPW, NPW)], sem)
            for t in range(16)
        ]
        for cp in copies:
            cp.wait()

    def rbody(i, _):
        sl = pl.ds(i * L, L)
        acc = buf_v[sl]
        for t in range(1, NW):
            acc = acc + buf_v[pl.ds(t * NPW + i * L, L)]
        r_v[sl] = 1.0 / (acc + 1e-16)
        return 0
    lax.fori_loop(0, NPW // L, rbody, 0)
    pltpu.sync_copy(r_v, rden_h.at[pl.ds(base, NPW)])


def _k1b(dparts):
    return pl.kernel(
        _k1b_body,
        out_type=_f32(NP),
        mesh=_mesh(),
        scratch_types=[
            pltpu.VMEM((NW * NPW,), jnp.float32),
            pltpu.VMEM((NPW,), jnp.float32),
            pltpu.SemaphoreType.DMA,
        ],
        **_SC_PARAMS,
    )(dparts.reshape(NW * NP))


def _k2_body(nblk, xp_h, pk_h, ex_h, rden_h, out_h,
             xp_v, out_v, rden_v, pk_c0, ex_c0, pk_c1, ex_c1, sem0, sem1):
    c = lax.axis_index("c")
    s = lax.axis_index("s")
    wid = s * NC + c
    npasses = nblk // NW
    iota = lax.iota(jnp.int32, L)
    pat = lax.shift_right_logical(iota, 2)
    lanem = jnp.bitwise_and(iota, 3)
    zero = jnp.zeros((L,), jnp.float32)
    nchunks = EP // EPW

    def start(off, pk_c, ex_c, sem):
        pltpu.async_copy(pk_h.at[pl.ds(off, EPW)], pk_c, sem)
        pltpu.async_copy(ex_h.at[pl.ds(off, EPW)], ex_c, sem)

    def drain(pk_c, ex_c, sem):
        pltpu.make_async_copy(pk_h.at[pl.ds(0, EPW)], pk_c, sem).wait()
        pltpu.make_async_copy(ex_h.at[pl.ds(0, EPW)], ex_c, sem).wait()

    rep_idx = [k * 4 + pat for k in range(4)]

    def compute(pk_c, ex_c):
        @plsc.parallel_loop(0, EPW // 16, unroll=4)
        def gbody(g):
            for k in range(4):
                idx = g * 16 + rep_idx[k]
                pk = plsc.load_gather(pk_c, [idx])
                xrep = plsc.load_gather(ex_c, [idx])
                sa = jnp.bitwise_and(pk, 65535) + lanem
                da = lax.shift_right_logical(pk, 16) + lanem
                vals = plsc.load_gather(xp_v, [sa])
                plsc.addupdate_scatter(out_v, [da], vals * xrep)

    pltpu.sync_copy(rden_h, rden_v)
    for p in range(npasses):
        fbi = p * NW + wid
        pltpu.sync_copy(xp_h.at[pl.ds(fbi * NP4, NP4)], out_v)
        for j in range(4):
            @plsc.parallel_loop(0, NP // L, unroll=4)
            def tbody(i, j=j):
                v = out_v[pl.ds(j * NP + i * L, L)]
                plsc.store_scatter(xp_v, [(i * L + iota) * 4 + j], v)

        @plsc.parallel_loop(0, NP4 // L, unroll=4)
        def zbody(i):
            out_v[pl.ds(i * L, L)] = zero

        start(0, pk_c0, ex_c0, sem0)

        def cbody(i, _):
            start((2 * i + 1) * EPW, pk_c1, ex_c1, sem1)
            drain(pk_c0, ex_c0, sem0)
            compute(pk_c0, ex_c0)
            nxt = jnp.minimum((2 * i + 2) * EPW, EP - EPW)
            start(nxt, pk_c0, ex_c0, sem0)
            drain(pk_c1, ex_c1, sem1)
            compute(pk_c1, ex_c1)
            return 0
        lax.fori_loop(0, nchunks // 2, cbody, 0)
        drain(pk_c0, ex_c0, sem0)

        for j in range(4):
            @plsc.parallel_loop(0, NP // L, unroll=4)
            def ubody(i, j=j):
                vals = plsc.load_gather(out_v, [(i * L + iota) * 4 + j])
                xp_v[pl.ds(j * NP + i * L, L)] = vals * rden_v[pl.ds(i * L, L)]
        pltpu.sync_copy(xp_v, out_h.at[pl.ds(fbi * NP4, NP4)])


def _k2(xpt_flat, pk, ex, rden, d):
    nblk = d // 4
    return pl.kernel(
        functools.partial(_k2_body, nblk),
        out_type=_f32(nblk * NP4),
        mesh=_mesh(),
        scratch_types=[
            pltpu.VMEM((NP4,), jnp.float32),
            pltpu.VMEM((NP4,), jnp.float32),
            pltpu.VMEM((NP,), jnp.float32),
            pltpu.VMEM((EPW,), jnp.int32),
            pltpu.VMEM((EPW,), jnp.float32),
            pltpu.VMEM((EPW,), jnp.int32),
            pltpu.VMEM((EPW,), jnp.float32),
            pltpu.SemaphoreType.DMA,
            pltpu.SemaphoreType.DMA,
        ],
        **_SC_PARAMS,
    )(xpt_flat, pk, ex, rden)


def _gat_sc(ht, W, a_s, a_d, bias, act, pk):
    d = W.shape[1]
    xpt, als, ald = _mm_xp(ht, W.T, a_s, a_d, bias, act)
    ex, dparts = _k1(als, ald, pk)
    rden = _k1b(dparts)
    out_t = _k2(xpt.reshape(-1), pk, ex, rden, d)
    return out_t.reshape(d, NP)


def kernel(X_v, edge_index, W_g0a, asrc_g0a, adst_g0a, bias_g0a, W_g0b,
           asrc_g0b, adst_g0b, bias_g0b, lin_W0, lin_b0, W_g1a, asrc_g1a,
           adst_g1a, bias_g1a, W_g1b, asrc_g1b, adst_g1b, bias_g1b, lin_W1,
           lin_b1, W_out, b_out):
    loops = jnp.arange(N, dtype=jnp.int32)
    pad = jnp.full((EP - E_TOT,), N, dtype=jnp.int32)
    src = jnp.concatenate([edge_index[0].astype(jnp.int32), loops, pad])
    dst = jnp.concatenate([edge_index[1].astype(jnp.int32), loops, pad])
    pk = jnp.bitwise_or(jnp.left_shift(src, 2), jnp.left_shift(dst, 18))

    ht = jnp.pad(X_v, ((0, NP - N), (0, 0))).T
    params = (
        (W_g0a, asrc_g0a, adst_g0a, W_g0b, asrc_g0b, adst_g0b,
         bias_g0a, bias_g0b, lin_W0, lin_b0),
        (W_g1a, asrc_g1a, adst_g1a, W_g1b, asrc_g1b, adst_g1b,
         bias_g1a, bias_g1b, lin_W1, lin_b1),
    )
    for l in range(2):
        (Wa, asa, ada, Wb, asb, adb, ba, bb, lin_W, lin_b) = params[l]
        d = Wa.shape[0]
        zba = jnp.zeros_like(ba)
        out_a = _gat_sc(ht, Wa, asa, ada, zba, "none", pk)
        out_b = _gat_sc(out_a, Wb, asb, adb, ba, "relu", pk)
        W1 = lin_W[:, :d]
        W2 = lin_W[:, d:]
        ht = _mm_lin(ht, out_b, bb, W1, W2, lin_b)
    return _mm_final(ht, W_out.T, b_out)


# speedup vs baseline: 1.0385x; 1.0385x over previous
import functools

import jax
import jax.numpy as jnp
from jax import lax
from jax.experimental import pallas as pl
from jax.experimental.pallas import tpu as pltpu
from jax.experimental.pallas import tpu_sc as plsc

N = 10000
NP = 10240
E_RAW = 160000
E_TOT = E_RAW + N
EP = 170496
NC, NS, L = 2, 16, 16
NW = NC * NS
EPW = EP // NW
NPW = NP // NW
RB = 256
NRB = NP // RB
NP4 = NP * 4

_mesh_cache = []


def _mesh():
    if not _mesh_cache:
        _mesh_cache.append(plsc.VectorSubcoreMesh(
            core_axis_name="c", subcore_axis_name="s",
            num_cores=NC, num_subcores=NS))
    return _mesh_cache[0]


def _f32(*shape):
    return jax.ShapeDtypeStruct(shape, jnp.float32)


_SC_PARAMS = dict(compiler_params=pltpu.CompilerParams(
    needs_layout_passes=False))


def _mm_xp_body(act, *refs):
    if act == "relu":
        (ht_ref, dp_ref, wt_ref, as_ref, ad_ref, b_ref, xpt_ref,
         als_ref, ald_ref) = refs
        dsum = jnp.sum(dp_ref[...], axis=0, keepdims=True) + 1e-16
        ht = jnp.maximum(ht_ref[...] / dsum + b_ref[...], 0.0)
    else:
        (ht_ref, wt_ref, as_ref, ad_ref, b_ref, xpt_ref,
         als_ref, ald_ref) = refs
        ht = ht_ref[...]
    xpt = jnp.dot(wt_ref[...], ht, preferred_element_type=jnp.float32)
    xpt_ref[...] = xpt
    als_ref[0, 0, :] = jnp.dot(as_ref[...], xpt,
                               preferred_element_type=jnp.float32)[0]
    ald_ref[0, 0, :] = jnp.dot(ad_ref[...], xpt,
                               preferred_element_type=jnp.float32)[0]


def _mm_xp(ht, WT, a_s, a_d, bias, act, dparts=None):
    d_out, d_in = WT.shape
    in_specs = [pl.BlockSpec((d_in, RB), lambda i: (0, i))]
    args = [ht]
    if act == "relu":
        in_specs.append(pl.BlockSpec((NW, RB), lambda i: (0, i)))
        args.append(dparts)
    in_specs += [
        pl.BlockSpec((d_out, d_in), lambda i: (0, 0)),
        pl.BlockSpec((1, d_out), lambda i: (0, 0)),
        pl.BlockSpec((1, d_out), lambda i: (0, 0)),
        pl.BlockSpec((d_in, 1), lambda i: (0, 0)),
    ]
    args += [WT, a_s[None, :], a_d[None, :], bias[:, None]]
    out = pl.pallas_call(
        functools.partial(_mm_xp_body, act),
        grid=(NRB,),
        in_specs=in_specs,
        out_specs=[
            pl.BlockSpec((d_out, RB), lambda i: (0, i)),
            pl.BlockSpec((1, 1, RB), lambda i: (i, 0, 0)),
            pl.BlockSpec((1, 1, RB), lambda i: (i, 0, 0)),
        ],
        out_shape=[_f32(d_out, NP), _f32(NRB, 1, RB), _f32(NRB, 1, RB)],
    )(*args)
    xpt, als3, ald3 = out
    return xpt, als3.reshape(NP), ald3.reshape(NP)


def _mm_lin_body(hp_ref, ob_ref, dp_ref, bb_ref, w1_ref, w2_ref, lb_ref,
                 o_ref):
    dsum = jnp.sum(dp_ref[...], axis=0, keepdims=True) + 1e-16
    t = jnp.tanh(ob_ref[...] / dsum + bb_ref[...])
    acc = jnp.dot(w1_ref[...], hp_ref[...], preferred_element_type=jnp.float32)
    acc += jnp.dot(w2_ref[...], t, preferred_element_type=jnp.float32)
    o_ref[...] = jnp.maximum(acc + lb_ref[...], 0.0)


def _mm_lin(hp_t, ob_t, dparts, bias_b, W1, W2, lin_b):
    hl, dp = W1.shape
    d = W2.shape[1]
    return pl.pallas_call(
        _mm_lin_body,
        grid=(NRB,),
        in_specs=[
            pl.BlockSpec((dp, RB), lambda i: (0, i)),
            pl.BlockSpec((d, RB), lambda i: (0, i)),
            pl.BlockSpec((NW, RB), lambda i: (0, i)),
            pl.BlockSpec((d, 1), lambda i: (0, 0)),
            pl.BlockSpec((hl, dp), lambda i: (0, 0)),
            pl.BlockSpec((hl, d), lambda i: (0, 0)),
            pl.BlockSpec((hl, 1), lambda i: (0, 0)),
        ],
        out_specs=pl.BlockSpec((hl, RB), lambda i: (0, i)),
        out_shape=_f32(hl, NP),
    )(hp_t, ob_t, dparts, bias_b[:, None], W1, W2, lin_b[:, None])


def _mm_final_body(ht_ref, wt_ref, b_ref, o_ref):
    col = ht_ref[:, pl.ds(14, 1)]
    o_ref[0, :] = jnp.sum(wt_ref[...] * col, axis=0) + b_ref[0, :]


def _mm_final(ht, WoutT, b_out):
    hl, o = WoutT.shape
    out = pl.pallas_call(
        _mm_final_body,
        grid=(1,),
        in_specs=[
            pl.BlockSpec((hl, 128), lambda i: (0, 0)),
            pl.BlockSpec((hl, o), lambda i: (0, 0)),
            pl.BlockSpec((1, o), lambda i: (0, 0)),
        ],
        out_specs=pl.BlockSpec((1, o), lambda i: (0, 0)),
        out_shape=_f32(1, o),
    )(ht, WoutT, b_out[None, :])
    return out[0]


def _k1_body(als_h, ald_h, pk_h, ex_h, dp_h,
             als_v, ald_v, pk_v, ex_v, dp_v):
    c = lax.axis_index("c")
    s = lax.axis_index("s")
    wid = s * NC + c
    base = wid * EPW
    pltpu.sync_copy(als_h, als_v)
    pltpu.sync_copy(ald_h, ald_v)
    pltpu.sync_copy(pk_h.at[pl.ds(base, EPW)], pk_v)

    zero = jnp.zeros((L,), jnp.float32)

    def zbody(i, _):
        dp_v[pl.ds(i * L, L)] = zero
        return 0
    lax.fori_loop(0, NP // L, zbody, 0)

    def ebody(g, _):
        sl = pl.ds(g * L, L)
        pk = pk_v[sl]
        s16 = lax.shift_right_logical(jnp.bitwise_and(pk, 65535), 2)
        d16 = lax.shift_right_logical(pk, 18)
        e = (plsc.load_gather(als_v, [s16])
             + plsc.load_gather(ald_v, [d16]))
        e = jnp.where(e >= 0.0, e, e * 0.2)
        ex = jnp.exp(e)
        ex_v[sl] = ex
        plsc.addupdate_scatter(dp_v, [d16], ex)
        return 0
    lax.fori_loop(0, EPW // L, ebody, 0)

    pltpu.sync_copy(ex_v, ex_h.at[pl.ds(base, EPW)])
    pltpu.sync_copy(dp_v, dp_h.at[wid])


def _k1(als, ald, pk):
    return pl.kernel(
        _k1_body,
        out_type=(_f32(EP), _f32(NW, NP)),
        mesh=_mesh(),
        scratch_types=[
            pltpu.VMEM((NP,), jnp.float32),
            pltpu.VMEM((NP,), jnp.float32),
            pltpu.VMEM((EPW,), jnp.int32),
            pltpu.VMEM((EPW,), jnp.float32),
            pltpu.VMEM((NP,), jnp.float32),
        ],
        **_SC_PARAMS,
    )(als, ald, pk)


def _k2_body(nblk, xp_h, pk_h, ex_h, out_h,
             xp_v, out_v, pk_c0, ex_c0, pk_c1, ex_c1, sem0, sem1):
    c = lax.axis_index("c")
    s = lax.axis_index("s")
    wid = s * NC + c
    npasses = nblk // NW
    iota = lax.iota(jnp.int32, L)
    pat = lax.shift_right_logical(iota, 2)
    lanem = jnp.bitwise_and(iota, 3)
    zero = jnp.zeros((L,), jnp.float32)
    nchunks = EP // EPW

    def start(off, pk_c, ex_c, sem):
        pltpu.async_copy(pk_h.at[pl.ds(off, EPW)], pk_c, sem)
        pltpu.async_copy(ex_h.at[pl.ds(off, EPW)], ex_c, sem)

    def drain(pk_c, ex_c, sem):
        pltpu.make_async_copy(pk_h.at[pl.ds(0, EPW)], pk_c, sem).wait()
        pltpu.make_async_copy(ex_h.at[pl.ds(0, EPW)], ex_c, sem).wait()

    rep_idx = [k * 4 + pat for k in range(4)]

    def compute(pk_c, ex_c):
        @plsc.parallel_loop(0, EPW // 16, unroll=4)
        def gbody(g):
            for k in range(4):
                idx = g * 16 + rep_idx[k]
                pk = plsc.load_gather(pk_c, [idx])
                xrep = plsc.load_gather(ex_c, [idx])
                sa = jnp.bitwise_and(pk, 65535) + lanem
                da = lax.shift_right_logical(pk, 16) + lanem
                vals = plsc.load_gather(xp_v, [sa])
                plsc.addupdate_scatter(out_v, [da], vals * xrep)

    for p in range(npasses):
        fbi = p * NW + wid
        pltpu.sync_copy(xp_h.at[pl.ds(fbi * NP4, NP4)], out_v)
        for j in range(4):
            @plsc.parallel_loop(0, NP // L, unroll=4)
            def tbody(i, j=j):
                v = out_v[pl.ds(j * NP + i * L, L)]
                plsc.store_scatter(xp_v, [(i * L + iota) * 4 + j], v)

        @plsc.parallel_loop(0, NP4 // L, unroll=4)
        def zbody(i):
            out_v[pl.ds(i * L, L)] = zero

        start(0, pk_c0, ex_c0, sem0)

        def cbody(i, _):
            start((2 * i + 1) * EPW, pk_c1, ex_c1, sem1)
            drain(pk_c0, ex_c0, sem0)
            compute(pk_c0, ex_c0)
            nxt = jnp.minimum((2 * i + 2) * EPW, EP - EPW)
            start(nxt, pk_c0, ex_c0, sem0)
            drain(pk_c1, ex_c1, sem1)
            compute(pk_c1, ex_c1)
            return 0
        lax.fori_loop(0, nchunks // 2, cbody, 0)
        drain(pk_c0, ex_c0, sem0)

        for j in range(4):
            @plsc.parallel_loop(0, NP // L, unroll=4)
            def ubody(i, j=j):
                vals = plsc.load_gather(out_v, [(i * L + iota) * 4 + j])
                xp_v[pl.ds(j * NP + i * L, L)] = vals
        pltpu.sync_copy(xp_v, out_h.at[pl.ds(fbi * NP4, NP4)])


def _k2(xpt_flat, pk, ex, d):
    nblk = d // 4
    return pl.kernel(
        functools.partial(_k2_body, nblk),
        out_type=_f32(nblk * NP4),
        mesh=_mesh(),
        scratch_types=[
            pltpu.VMEM((NP4,), jnp.float32),
            pltpu.VMEM((NP4,), jnp.float32),
            pltpu.VMEM((EPW,), jnp.int32),
            pltpu.VMEM((EPW,), jnp.float32),
            pltpu.VMEM((EPW,), jnp.int32),
            pltpu.VMEM((EPW,), jnp.float32),
            pltpu.SemaphoreType.DMA,
            pltpu.SemaphoreType.DMA,
        ],
        **_SC_PARAMS,
    )(xpt_flat, pk, ex)


def _gat_sc(ht, W, a_s, a_d, bias, act, pk, dparts_in=None):
    d = W.shape[1]
    xpt, als, ald = _mm_xp(ht, W.T, a_s, a_d, bias, act, dparts_in)
    ex, dparts = _k1(als, ald, pk)
    out_t = _k2(xpt.reshape(-1), pk, ex, d)
    return out_t.reshape(d, NP), dparts


def kernel(X_v, edge_index, W_g0a, asrc_g0a, adst_g0a, bias_g0a, W_g0b,
           asrc_g0b, adst_g0b, bias_g0b, lin_W0, lin_b0, W_g1a, asrc_g1a,
           adst_g1a, bias_g1a, W_g1b, asrc_g1b, adst_g1b, bias_g1b, lin_W1,
           lin_b1, W_out, b_out):
    loops = jnp.arange(N, dtype=jnp.int32)
    pad = jnp.full((EP - E_TOT,), N, dtype=jnp.int32)
    src = jnp.concatenate([edge_index[0].astype(jnp.int32), loops, pad])
    dst = jnp.concatenate([edge_index[1].astype(jnp.int32), loops, pad])
    pk = jnp.bitwise_or(jnp.left_shift(src, 2), jnp.left_shift(dst, 18))

    ht = jnp.pad(X_v, ((0, NP - N), (0, 0))).T
    params = (
        (W_g0a, asrc_g0a, adst_g0a, W_g0b, asrc_g0b, adst_g0b,
         bias_g0a, bias_g0b, lin_W0, lin_b0),
        (W_g1a, asrc_g1a, adst_g1a, W_g1b, asrc_g1b, adst_g1b,
         bias_g1a, bias_g1b, lin_W1, lin_b1),
    )
    for l in range(2):
        (Wa, asa, ada, Wb, asb, adb, ba, bb, lin_W, lin_b) = params[l]
        d = Wa.shape[0]
        zba = jnp.zeros_like(ba)
        out_a, dp_a = _gat_sc(ht, Wa, asa, ada, zba, "none", pk)
        out_b, dp_b = _gat_sc(out_a, Wb, asb, adb, ba, "relu", pk, dp_a)
        W1 = lin_W[:, :d]
        W2 = lin_W[:, d:]
        ht = _mm_lin(ht, out_b, dp_b, bb, W1, W2, lin_b)
    return _mm_final(ht, W_out.T, b_out)


# speedup vs baseline: 1.0581x; 1.0189x over previous
import functools

import jax
import jax.numpy as jnp
from jax import lax
from jax.experimental import pallas as pl
from jax.experimental.pallas import tpu as pltpu
from jax.experimental.pallas import tpu_sc as plsc

N = 10000
NP = 10240
E_RAW = 160000
E_TOT = E_RAW + N
EP = 170496
NC, NS, L = 2, 16, 16
NW = NC * NS
EPW = EP // NW
NPW = NP // NW
RB = 256
NRB = NP // RB
NP4 = NP * 4
CH = EP // 16

_mesh_cache = []


def _mesh():
    if not _mesh_cache:
        _mesh_cache.append(plsc.VectorSubcoreMesh(
            core_axis_name="c", subcore_axis_name="s",
            num_cores=NC, num_subcores=NS))
    return _mesh_cache[0]


def _f32(*shape):
    return jax.ShapeDtypeStruct(shape, jnp.float32)


_SC_PARAMS = dict(compiler_params=pltpu.CompilerParams(
    needs_layout_passes=False))


def _mm_xp_body(act, *refs):
    if act == "relu":
        (ht_ref, dp_ref, wt_ref, as_ref, ad_ref, b_ref, xpt_ref,
         als_ref, ald_ref) = refs
        dsum = jnp.sum(dp_ref[...], axis=0, keepdims=True) + 1e-16
        ht = jnp.maximum(ht_ref[...] / dsum + b_ref[...], 0.0)
    else:
        (ht_ref, wt_ref, as_ref, ad_ref, b_ref, xpt_ref,
         als_ref, ald_ref) = refs
        ht = ht_ref[...]
    xpt = jnp.dot(wt_ref[...], ht, preferred_element_type=jnp.float32)
    xpt_ref[...] = xpt
    als_ref[0, 0, :] = jnp.dot(as_ref[...], xpt,
                               preferred_element_type=jnp.float32)[0]
    ald_ref[0, 0, :] = jnp.dot(ad_ref[...], xpt,
                               preferred_element_type=jnp.float32)[0]


def _mm_xp(ht, WT, a_s, a_d, bias, act, dparts=None):
    d_out, d_in = WT.shape
    in_specs = [pl.BlockSpec((d_in, RB), lambda i: (0, i))]
    args = [ht]
    if act == "relu":
        in_specs.append(pl.BlockSpec((NW, RB), lambda i: (0, i)))
        args.append(dparts)
    in_specs += [
        pl.BlockSpec((d_out, d_in), lambda i: (0, 0)),
        pl.BlockSpec((1, d_out), lambda i: (0, 0)),
        pl.BlockSpec((1, d_out), lambda i: (0, 0)),
        pl.BlockSpec((d_in, 1), lambda i: (0, 0)),
    ]
    args += [WT, a_s[None, :], a_d[None, :], bias[:, None]]
    out = pl.pallas_call(
        functools.partial(_mm_xp_body, act),
        grid=(NRB,),
        in_specs=in_specs,
        out_specs=[
            pl.BlockSpec((d_out, RB), lambda i: (0, i)),
            pl.BlockSpec((1, 1, RB), lambda i: (i, 0, 0)),
            pl.BlockSpec((1, 1, RB), lambda i: (i, 0, 0)),
        ],
        out_shape=[_f32(d_out, NP), _f32(NRB, 1, RB), _f32(NRB, 1, RB)],
    )(*args)
    xpt, als3, ald3 = out
    return xpt, als3.reshape(NP), ald3.reshape(NP)


def _mm_lin_body(hp_ref, ob_ref, dp_ref, bb_ref, w1_ref, w2_ref, lb_ref,
                 o_ref):
    dsum = jnp.sum(dp_ref[...], axis=0, keepdims=True) + 1e-16
    t = jnp.tanh(ob_ref[...] / dsum + bb_ref[...])
    acc = jnp.dot(w1_ref[...], hp_ref[...], preferred_element_type=jnp.float32)
    acc += jnp.dot(w2_ref[...], t, preferred_element_type=jnp.float32)
    o_ref[...] = jnp.maximum(acc + lb_ref[...], 0.0)


def _mm_lin(hp_t, ob_t, dparts, bias_b, W1, W2, lin_b):
    hl, dp = W1.shape
    d = W2.shape[1]
    return pl.pallas_call(
        _mm_lin_body,
        grid=(NRB,),
        in_specs=[
            pl.BlockSpec((dp, RB), lambda i: (0, i)),
            pl.BlockSpec((d, RB), lambda i: (0, i)),
            pl.BlockSpec((NW, RB), lambda i: (0, i)),
            pl.BlockSpec((d, 1), lambda i: (0, 0)),
            pl.BlockSpec((hl, dp), lambda i: (0, 0)),
            pl.BlockSpec((hl, d), lambda i: (0, 0)),
            pl.BlockSpec((hl, 1), lambda i: (0, 0)),
        ],
        out_specs=pl.BlockSpec((hl, RB), lambda i: (0, i)),
        out_shape=_f32(hl, NP),
    )(hp_t, ob_t, dparts, bias_b[:, None], W1, W2, lin_b[:, None])


def _mm_final_body(ht_ref, wt_ref, b_ref, o_ref):
    col = ht_ref[:, pl.ds(14, 1)]
    o_ref[0, :] = jnp.sum(wt_ref[...] * col, axis=0) + b_ref[0, :]


def _mm_final(ht, WoutT, b_out):
    hl, o = WoutT.shape
    out = pl.pallas_call(
        _mm_final_body,
        grid=(1,),
        in_specs=[
            pl.BlockSpec((hl, 128), lambda i: (0, 0)),
            pl.BlockSpec((hl, o), lambda i: (0, 0)),
            pl.BlockSpec((1, o), lambda i: (0, 0)),
        ],
        out_specs=pl.BlockSpec((1, o), lambda i: (0, 0)),
        out_shape=_f32(1, o),
    )(ht, WoutT, b_out[None, :])
    return out[0]


def _k1_body(als_h, ald_h, pk_h, ex_h, dp_h,
             als_v, ald_v, pk_v, ex_v, dp_v):
    c = lax.axis_index("c")
    s = lax.axis_index("s")
    wid = s * NC + c
    base = wid * EPW
    pltpu.sync_copy(als_h, als_v)
    pltpu.sync_copy(ald_h, ald_v)
    pltpu.sync_copy(pk_h.at[pl.ds(base, EPW)], pk_v)

    zero = jnp.zeros((L,), jnp.float32)

    @plsc.parallel_loop(0, NP // L, unroll=4)
    def zbody(i):
        dp_v[pl.ds(i * L, L)] = zero

    @plsc.parallel_loop(0, EPW // L, unroll=2)
    def ebody(g):
        sl = pl.ds(g * L, L)
        pk = pk_v[sl]
        s16 = lax.shift_right_logical(jnp.bitwise_and(pk, 65535), 2)
        d16 = lax.shift_right_logical(pk, 18)
        e = (plsc.load_gather(als_v, [s16])
             + plsc.load_gather(ald_v, [d16]))
        e = jnp.where(e >= 0.0, e, e * 0.2)
        ex = jnp.exp(e)
        ex_v[sl] = ex
        plsc.addupdate_scatter(dp_v, [d16], ex)

    pltpu.sync_copy(ex_v, ex_h.at[pl.ds(base, EPW)])
    pltpu.sync_copy(dp_v, dp_h.at[wid])


def _k1(als, ald, pk):
    return pl.kernel(
        _k1_body,
        out_type=(_f32(EP), _f32(NW, NP)),
        mesh=_mesh(),
        scratch_types=[
            pltpu.VMEM((NP,), jnp.float32),
            pltpu.VMEM((NP,), jnp.float32),
            pltpu.VMEM((EPW,), jnp.int32),
            pltpu.VMEM((EPW,), jnp.float32),
            pltpu.VMEM((NP,), jnp.float32),
        ],
        **_SC_PARAMS,
    )(als, ald, pk)


def _k2_body(nblk, xp_h, pk_h, ex_h, out_h,
             xp_v, out_v, pk_c0, ex_c0, pk_c1, ex_c1, sem0, sem1):
    c = lax.axis_index("c")
    s = lax.axis_index("s")
    wid = s * NC + c
    npasses = nblk // NW
    iota = lax.iota(jnp.int32, L)
    pat = lax.shift_right_logical(iota, 2)
    lanem = jnp.bitwise_and(iota, 3)
    zero = jnp.zeros((L,), jnp.float32)
    nchunks = EP // CH

    def start(off, pk_c, ex_c, sem):
        pltpu.async_copy(pk_h.at[pl.ds(off, CH)], pk_c, sem)
        pltpu.async_copy(ex_h.at[pl.ds(off, CH)], ex_c, sem)

    def drain(pk_c, ex_c, sem):
        pltpu.make_async_copy(pk_h.at[pl.ds(0, CH)], pk_c, sem).wait()
        pltpu.make_async_copy(ex_h.at[pl.ds(0, CH)], ex_c, sem).wait()

    rep_idx = [k * 4 + pat for k in range(4)]

    def compute(pk_c, ex_c):
        @plsc.parallel_loop(0, CH // 16, unroll=4)
        def gbody(g):
            for k in range(4):
                idx = g * 16 + rep_idx[k]
                pk = plsc.load_gather(pk_c, [idx])
                xrep = plsc.load_gather(ex_c, [idx])
                sa = jnp.bitwise_and(pk, 65535) + lanem
                da = lax.shift_right_logical(pk, 16) + lanem
                vals = plsc.load_gather(xp_v, [sa])
                plsc.addupdate_scatter(out_v, [da], vals * xrep)

    for p in range(npasses):
        fbi = p * NW + wid
        pltpu.sync_copy(xp_h.at[pl.ds(fbi * NP4, NP4)], out_v)
        for j in range(4):
            @plsc.parallel_loop(0, NP // L, unroll=4)
            def tbody(i, j=j):
                v = out_v[pl.ds(j * NP + i * L, L)]
                plsc.store_scatter(xp_v, [(i * L + iota) * 4 + j], v)

        @plsc.parallel_loop(0, NP4 // L, unroll=4)
        def zbody(i):
            out_v[pl.ds(i * L, L)] = zero

        start(0, pk_c0, ex_c0, sem0)

        def cbody(i, _):
            start((2 * i + 1) * CH, pk_c1, ex_c1, sem1)
            drain(pk_c0, ex_c0, sem0)
            compute(pk_c0, ex_c0)
            nxt = jnp.minimum((2 * i + 2) * CH, EP - CH)
            start(nxt, pk_c0, ex_c0, sem0)
            drain(pk_c1, ex_c1, sem1)
            compute(pk_c1, ex_c1)
            return 0
        lax.fori_loop(0, nchunks // 2, cbody, 0)
        drain(pk_c0, ex_c0, sem0)

        for j in range(4):
            @plsc.parallel_loop(0, NP // L, unroll=4)
            def ubody(i, j=j):
                vals = plsc.load_gather(out_v, [(i * L + iota) * 4 + j])
                xp_v[pl.ds(j * NP + i * L, L)] = vals
        pltpu.sync_copy(xp_v, out_h.at[pl.ds(fbi * NP4, NP4)])


def _k2(xpt_flat, pk, ex, d):
    nblk = d // 4
    return pl.kernel(
        functools.partial(_k2_body, nblk),
        out_type=_f32(nblk * NP4),
        mesh=_mesh(),
        scratch_types=[
            pltpu.VMEM((NP4,), jnp.float32),
            pltpu.VMEM((NP4,), jnp.float32),
            pltpu.VMEM((CH,), jnp.int32),
            pltpu.VMEM((CH,), jnp.float32),
            pltpu.VMEM((CH,), jnp.int32),
            pltpu.VMEM((CH,), jnp.float32),
            pltpu.SemaphoreType.DMA,
            pltpu.SemaphoreType.DMA,
        ],
        **_SC_PARAMS,
    )(xpt_flat, pk, ex)


def _gat_sc(ht, W, a_s, a_d, bias, act, pk, dparts_in=None):
    d = W.shape[1]
    xpt, als, ald = _mm_xp(ht, W.T, a_s, a_d, bias, act, dparts_in)
    ex, dparts = _k1(als, ald, pk)
    out_t = _k2(xpt.reshape(-1), pk, ex, d)
    return out_t.reshape(d, NP), dparts


def kernel(X_v, edge_index, W_g0a, asrc_g0a, adst_g0a, bias_g0a, W_g0b,
           asrc_g0b, adst_g0b, bias_g0b, lin_W0, lin_b0, W_g1a, asrc_g1a,
           adst_g1a, bias_g1a, W_g1b, asrc_g1b, adst_g1b, bias_g1b, lin_W1,
           lin_b1, W_out, b_out):
    loops = jnp.arange(N, dtype=jnp.int32)
    pad = jnp.full((EP - E_TOT,), N, dtype=jnp.int32)
    src = jnp.concatenate([edge_index[0].astype(jnp.int32), loops, pad])
    dst = jnp.concatenate([edge_index[1].astype(jnp.int32), loops, pad])
    pk = jnp.bitwise_or(jnp.left_shift(src, 2), jnp.left_shift(dst, 18))

    ht = jnp.pad(X_v, ((0, NP - N), (0, 0))).T
    params = (
        (W_g0a, asrc_g0a, adst_g0a, W_g0b, asrc_g0b, adst_g0b,
         bias_g0a, bias_g0b, lin_W0, lin_b0),
        (W_g1a, asrc_g1a, adst_g1a, W_g1b, asrc_g1b, adst_g1b,
         bias_g1a, bias_g1b, lin_W1, lin_b1),
    )
    for l in range(2):
        (Wa, asa, ada, Wb, asb, adb, ba, bb, lin_W, lin_b) = params[l]
        d = Wa.shape[0]
        zba = jnp.zeros_like(ba)
        out_a, dp_a = _gat_sc(ht, Wa, asa, ada, zba, "none", pk)
        out_b, dp_b = _gat_sc(out_a, Wb, asb, adb, ba, "relu", pk, dp_a)
        W1 = lin_W[:, :d]
        W2 = lin_W[:, d:]
        ht = _mm_lin(ht, out_b, dp_b, bb, W1, W2, lin_b)
    return _mm_final(ht, W_out.T, b_out)
